# baseline (speedup 1.0000x reference)
import functools

import jax
import jax.numpy as jnp
from jax import lax
from jax.experimental import pallas as pl
from jax.experimental.pallas import tpu as pltpu
from jax.experimental.pallas import tpu_sc as plsc

N = 10000
E = 320000
D = 128
G = 64

NW = 32
CH = 96
NCH = 105
EPAD = NW * NCH * CH
TRASH = 10008
ROWS_T = 632
NP = 16 * ROWS_T
DEG_T = 640
NPD = 16 * DEG_T

BN = 1000
GRID = N // BN

_mesh = plsc.VectorSubcoreMesh(core_axis_name="c", subcore_axis_name="s")


@functools.partial(
    pl.kernel,
    mesh=_mesh,
    out_type=jax.ShapeDtypeStruct((2, NPD), jnp.float32),
    scratch_types=[
        pltpu.VMEM((NCH, CH), jnp.int32),
        pltpu.VMEM((CH,), jnp.float32),
        pltpu.VMEM_SHARED((NPD,), jnp.float32),
    ],
)
def _sc_deg(dst_hbm, zt_hbm, ones_hbm, out_hbm, dst_v, ones_v, acc):
    c = lax.axis_index("c")
    t = lax.axis_index("s")
    wid = t * 2 + c
    pltpu.sync_copy(zt_hbm, acc.at[pl.ds(t * DEG_T, DEG_T)])
    pltpu.sync_copy(ones_hbm, ones_v)
    pltpu.sync_copy(dst_hbm.at[wid], dst_v)
    plsc.subcore_barrier()

    def body(j, carry):
        pltpu.sync_copy(ones_v, acc.at[dst_v.at[j]], add=True)
        return carry

    lax.fori_loop(0, NCH, body, 0)
    plsc.subcore_barrier()
    pltpu.sync_copy(acc.at[pl.ds(t * DEG_T, DEG_T)],
                    out_hbm.at[c, pl.ds(t * DEG_T, DEG_T)])


@functools.partial(
    pl.kernel,
    mesh=_mesh,
    out_type=jax.ShapeDtypeStruct((2, NP, D), jnp.float32),
    scratch_types=[
        pltpu.VMEM((NCH * CH,), jnp.int32),
        pltpu.VMEM((NCH, CH), jnp.int32),
        pltpu.VMEM((CH, D), jnp.float32),
        pltpu.VMEM_SHARED((NP, D), jnp.float32),
        pltpu.SemaphoreType.DMA,
    ],
)
def _sc_spmm(g_hbm, srcf_hbm, dst_hbm, zrows_hbm, out_hbm,
             src_v, dst_v, rows_v, acc, sem):
    c = lax.axis_index("c")
    t = lax.axis_index("s")
    wid = t * 2 + c
    pltpu.sync_copy(zrows_hbm, acc.at[pl.ds(t * ROWS_T, ROWS_T)])
    pltpu.sync_copy(srcf_hbm.at[wid], src_v)
    pltpu.sync_copy(dst_hbm.at[wid], dst_v)
    plsc.subcore_barrier()

    def body(j, carry):
        pltpu.async_copy(
            g_hbm.at[src_v.at[pl.ds(j * CH, CH)]], rows_v, sem).wait()
        pltpu.sync_copy(rows_v, acc.at[dst_v.at[j]], add=True)
        return carry

    lax.fori_loop(0, NCH, body, 0)
    plsc.subcore_barrier()
    pltpu.sync_copy(acc.at[pl.ds(t * ROWS_T, ROWS_T)],
                    out_hbm.at[c, pl.ds(t * ROWS_T, ROWS_T)])


def _k1_body(x_ref, w_ref, deg_ref, dinv_ref, g_ref):
    y = jnp.dot(x_ref[...], w_ref[...], preferred_element_type=jnp.float32)
    dsum = deg_ref[0] + deg_ref[1] + 1.0
    dinv = lax.rsqrt(dsum)
    dinv_ref[...] = dinv
    g_ref[...] = y * dinv


def _tc_prescale(x, W1, deg3):
    return pl.pallas_call(
        _k1_body,
        grid=(GRID,),
        in_specs=[
            pl.BlockSpec((BN, D), lambda i: (i, 0)),
            pl.BlockSpec((D, D), lambda i: (0, 0)),
            pl.BlockSpec((2, BN, 1), lambda i: (0, i, 0)),
        ],
        out_specs=[
            pl.BlockSpec((BN, 1), lambda i: (i, 0)),
            pl.BlockSpec((BN, D), lambda i: (i, 0)),
        ],
        out_shape=[
            jax.ShapeDtypeStruct((N, 1), jnp.float32),
            jax.ShapeDtypeStruct((N, D), jnp.float32),
        ],
    )(x, W1, deg3)


def _k2_body(s0_ref, s1_ref, g_ref, dinv_ref, b_ref, w_ref, gout_ref):
    ssum = s0_ref[0] + s1_ref[0] + g_ref[...]
    p = ssum * dinv_ref[...] + b_ref[...]
    h = jnp.maximum(p, 0.0)
    y = jnp.dot(h, w_ref[...], preferred_element_type=jnp.float32)
    gout_ref[...] = y * dinv_ref[...]


def _tc_layer(s, g, dinv, b_row, Wn):
    return pl.pallas_call(
        _k2_body,
        grid=(GRID,),
        in_specs=[
            pl.BlockSpec((1, BN, D), lambda i: (0, i, 0)),
            pl.BlockSpec((1, BN, D), lambda i: (1, i, 0)),
            pl.BlockSpec((BN, D), lambda i: (i, 0)),
            pl.BlockSpec((BN, 1), lambda i: (i, 0)),
            pl.BlockSpec((1, D), lambda i: (0, 0)),
            pl.BlockSpec((D, D), lambda i: (0, 0)),
        ],
        out_specs=pl.BlockSpec((BN, D), lambda i: (i, 0)),
        out_shape=jax.ShapeDtypeStruct((N, D), jnp.float32),
    )(s, s, g, dinv, b_row, Wn)


def _k4_body(s_ref, g_ref, dinv_ref, b3_ref, batch_ref,
             w1_ref, b1_ref, w2_ref, b2_ref, out_ref):
    h4 = ((s_ref[0, :N] + s_ref[1, :N] + g_ref[...]) * dinv_ref[...]
          + b3_ref[...])
    b = batch_ref[...]
    gids = lax.broadcasted_iota(jnp.int32, (G, N), 0)
    oh = (b[None, :] == gids).astype(jnp.float32)
    sums = jnp.dot(oh, h4, preferred_element_type=jnp.float32)
    counts = jnp.sum(oh, axis=1, keepdims=True)
    pooled = sums / jnp.maximum(counts, 1.0)
    hm = jnp.maximum(
        jnp.dot(pooled, w1_ref[...], preferred_element_type=jnp.float32)
        + b1_ref[...], 0.0)
    out_ref[...] = jnp.dot(
        hm, w2_ref[...], preferred_element_type=jnp.float32) + b2_ref[...]


def _tc_pool_mlp(s, g, dinv, b3_row, batch, mW1p, mb1p, mW2p, mb2p):
    return pl.pallas_call(
        _k4_body,
        out_shape=jax.ShapeDtypeStruct((G, D), jnp.float32),
    )(s, g, dinv, b3_row, batch, mW1p, mb1p, mW2p, mb2p)


def kernel(x, edge_index, batch, W1, b1, W2, b2, W3, b3, mW1, mb1, mW2, mb2):
    f32 = jnp.float32
    npad = EPAD - edge_index.shape[1]
    src_r = jnp.concatenate(
        [edge_index[0], jnp.zeros((npad,), jnp.int32)]).reshape(NW, NCH, CH)
    dst_r = jnp.concatenate(
        [edge_index[1], jnp.full((npad,), TRASH, jnp.int32)]).reshape(NW, NCH, CH)

    zt = jnp.zeros((DEG_T,), f32)
    ones_e = jnp.ones((CH,), f32)
    zrows = jnp.zeros((ROWS_T, D), f32)

    deg = _sc_deg(dst_r, zt, ones_e)
    deg3 = deg.reshape(2, NPD, 1)

    dinv, g1 = _tc_prescale(x.astype(f32), W1, deg3)

    src_f = src_r.reshape(NW, NCH * CH)
    s1 = _sc_spmm(g1, src_f, dst_r, zrows)
    g2 = _tc_layer(s1, g1, dinv, b1.reshape(1, D), W2)
    s2 = _sc_spmm(g2, src_f, dst_r, zrows)
    g3 = _tc_layer(s2, g2, dinv, b2.reshape(1, D), W3)
    s3 = _sc_spmm(g3, src_f, dst_r, zrows)

    mW1p = jnp.zeros((D, D), f32).at[:, :mW1.shape[1]].set(mW1)
    mb1p = jnp.zeros((1, D), f32).at[0, :mb1.shape[0]].set(mb1)
    mW2p = jnp.zeros((D, D), f32).at[:mW2.shape[0], :mW2.shape[1]].set(mW2)
    mb2p = jnp.zeros((1, D), f32).at[0, :mb2.shape[0]].set(mb2)

    pred_pad = _tc_pool_mlp(s3, g3, dinv, b3.reshape(1, D), batch,
                            mW1p, mb1p, mW2p, mb2p)
    return pred_pad[:, :mW2.shape[1]]


# speedup vs baseline: 1.3907x; 1.3907x over previous
import functools

import jax
import jax.numpy as jnp
from jax import lax
from jax.experimental import pallas as pl
from jax.experimental.pallas import tpu as pltpu
from jax.experimental.pallas import tpu_sc as plsc

N = 10000
E = 320000
D = 128
G = 64

NW = 32
CH = 80
NCH = 125
ROWS_T = 632
NP = 16 * ROWS_T
DEG_T = 640
NPD = 16 * DEG_T

BN = 1000
GRID = N // BN

_mesh = plsc.VectorSubcoreMesh(core_axis_name="c", subcore_axis_name="s")


@functools.partial(
    pl.kernel,
    mesh=_mesh,
    out_type=jax.ShapeDtypeStruct((2, NPD), jnp.float32),
    scratch_types=[
        pltpu.VMEM((NCH, CH), jnp.int32),
        pltpu.VMEM((CH,), jnp.float32),
        pltpu.VMEM_SHARED((NPD,), jnp.float32),
    ],
)
def _sc_deg(dst_hbm, zt_hbm, ones_hbm, out_hbm, dst_v, ones_v, acc):
    c = lax.axis_index("c")
    t = lax.axis_index("s")
    wid = t * 2 + c
    pltpu.sync_copy(zt_hbm, acc.at[pl.ds(t * DEG_T, DEG_T)])
    pltpu.sync_copy(ones_hbm, ones_v)
    pltpu.sync_copy(dst_hbm.at[wid], dst_v)
    plsc.subcore_barrier()

    def body(j, carry):
        pltpu.sync_copy(ones_v, acc.at[dst_v.at[j]], add=True)
        return carry

    lax.fori_loop(0, NCH, body, 0)
    plsc.subcore_barrier()
    pltpu.sync_copy(acc.at[pl.ds(t * DEG_T, DEG_T)],
                    out_hbm.at[c, pl.ds(t * DEG_T, DEG_T)])


@functools.partial(
    pl.kernel,
    mesh=_mesh,
    out_type=jax.ShapeDtypeStruct((2, NP, D), jnp.float32),
    scratch_types=[
        pltpu.VMEM((NCH, CH), jnp.int32),
        pltpu.VMEM((NCH, CH), jnp.int32),
        pltpu.VMEM((CH, D), jnp.float32),
        pltpu.VMEM_SHARED((NP, D), jnp.float32),
        pltpu.SemaphoreType.DMA,
    ],
)
def _sc_spmm(g_hbm, src_hbm, dst_hbm, zrows_hbm, out_hbm,
             src_v, dst_v, rows_v, acc, sem):
    c = lax.axis_index("c")
    t = lax.axis_index("s")
    wid = t * 2 + c
    pltpu.sync_copy(zrows_hbm, acc.at[pl.ds(t * ROWS_T, ROWS_T)])
    pltpu.sync_copy(src_hbm.at[wid], src_v)
    pltpu.sync_copy(dst_hbm.at[wid], dst_v)
    plsc.subcore_barrier()

    def body(j, carry):
        pltpu.async_copy(g_hbm.at[src_v.at[j]], rows_v, sem).wait()
        pltpu.sync_copy(rows_v, acc.at[dst_v.at[j]], add=True)
        return carry

    lax.fori_loop(0, NCH, body, 0)
    plsc.subcore_barrier()
    pltpu.sync_copy(acc.at[pl.ds(t * ROWS_T, ROWS_T)],
                    out_hbm.at[c, pl.ds(t * ROWS_T, ROWS_T)])


def _k1_body(x_ref, w_ref, deg_ref, dinv_ref, g_ref):
    y = jnp.dot(x_ref[...], w_ref[...], preferred_element_type=jnp.float32)
    dsum = deg_ref[0] + deg_ref[1] + 1.0
    dinv = lax.rsqrt(dsum)
    dinv_ref[...] = dinv
    g_ref[...] = y * dinv


def _tc_prescale(x, W1, deg3):
    return pl.pallas_call(
        _k1_body,
        grid=(GRID,),
        in_specs=[
            pl.BlockSpec((BN, D), lambda i: (i, 0)),
            pl.BlockSpec((D, D), lambda i: (0, 0)),
            pl.BlockSpec((2, BN, 1), lambda i: (0, i, 0)),
        ],
        out_specs=[
            pl.BlockSpec((BN, 1), lambda i: (i, 0)),
            pl.BlockSpec((BN, D), lambda i: (i, 0)),
        ],
        out_shape=[
            jax.ShapeDtypeStruct((N, 1), jnp.float32),
            jax.ShapeDtypeStruct((N, D), jnp.float32),
        ],
    )(x, W1, deg3)


def _k2_body(s0_ref, s1_ref, g_ref, dinv_ref, b_ref, w_ref, gout_ref):
    ssum = s0_ref[0] + s1_ref[0] + g_ref[...]
    p = ssum * dinv_ref[...] + b_ref[...]
    h = jnp.maximum(p, 0.0)
    y = jnp.dot(h, w_ref[...], preferred_element_type=jnp.float32)
    gout_ref[...] = y * dinv_ref[...]


def _tc_layer(s, g, dinv, b_row, Wn):
    return pl.pallas_call(
        _k2_body,
        grid=(GRID,),
        in_specs=[
            pl.BlockSpec((1, BN, D), lambda i: (0, i, 0)),
            pl.BlockSpec((1, BN, D), lambda i: (1, i, 0)),
            pl.BlockSpec((BN, D), lambda i: (i, 0)),
            pl.BlockSpec((BN, 1), lambda i: (i, 0)),
            pl.BlockSpec((1, D), lambda i: (0, 0)),
            pl.BlockSpec((D, D), lambda i: (0, 0)),
        ],
        out_specs=pl.BlockSpec((BN, D), lambda i: (i, 0)),
        out_shape=jax.ShapeDtypeStruct((N, D), jnp.float32),
    )(s, s, g, dinv, b_row, Wn)


def _k4_body(s_ref, g_ref, dinv_ref, b3_ref, batch_ref,
             w1_ref, b1_ref, w2_ref, b2_ref, out_ref):
    h4 = ((s_ref[0, :N] + s_ref[1, :N] + g_ref[...]) * dinv_ref[...]
          + b3_ref[...])
    b = batch_ref[...]
    gids = lax.broadcasted_iota(jnp.int32, (G, N), 0)
    oh = (b[None, :] == gids).astype(jnp.float32)
    sums = jnp.dot(oh, h4, preferred_element_type=jnp.float32)
    counts = jnp.sum(oh, axis=1, keepdims=True)
    pooled = sums / jnp.maximum(counts, 1.0)
    hm = jnp.maximum(
        jnp.dot(pooled, w1_ref[...], preferred_element_type=jnp.float32)
        + b1_ref[...], 0.0)
    out_ref[...] = jnp.dot(
        hm, w2_ref[...], preferred_element_type=jnp.float32) + b2_ref[...]


def _tc_pool_mlp(s, g, dinv, b3_row, batch, mW1p, mb1p, mW2p, mb2p):
    return pl.pallas_call(
        _k4_body,
        out_shape=jax.ShapeDtypeStruct((G, D), jnp.float32),
    )(s, g, dinv, b3_row, batch, mW1p, mb1p, mW2p, mb2p)


def kernel(x, edge_index, batch, W1, b1, W2, b2, W3, b3, mW1, mb1, mW2, mb2):
    f32 = jnp.float32
    src_r = edge_index[0].reshape(NW, NCH, CH)
    dst_r = edge_index[1].reshape(NW, NCH, CH)

    zt = jnp.zeros((DEG_T,), f32)
    ones_e = jnp.ones((CH,), f32)
    zrows = jnp.zeros((ROWS_T, D), f32)

    deg = _sc_deg(dst_r, zt, ones_e)
    deg3 = deg.reshape(2, NPD, 1)

    dinv, g1 = _tc_prescale(x.astype(f32), W1, deg3)

    s1 = _sc_spmm(g1, src_r, dst_r, zrows)
    g2 = _tc_layer(s1, g1, dinv, b1.reshape(1, D), W2)
    s2 = _sc_spmm(g2, src_r, dst_r, zrows)
    g3 = _tc_layer(s2, g2, dinv, b2.reshape(1, D), W3)
    s3 = _sc_spmm(g3, src_r, dst_r, zrows)

    mW1p = jnp.zeros((D, D), f32).at[:, :mW1.shape[1]].set(mW1)
    mb1p = jnp.zeros((1, D), f32).at[0, :mb1.shape[0]].set(mb1)
    mW2p = jnp.zeros((D, D), f32).at[:mW2.shape[0], :mW2.shape[1]].set(mW2)
    mb2p = jnp.zeros((1, D), f32).at[0, :mb2.shape[0]].set(mb2)

    pred_pad = _tc_pool_mlp(s3, g3, dinv, b3.reshape(1, D), batch,
                            mW1p, mb1p, mW2p, mb2p)
    return pred_pad[:, :mW2.shape[1]]
